# baseline (speedup 1.0000x reference)
import functools

import jax
import jax.numpy as jnp
from jax.experimental import pallas as pl
from jax.experimental.pallas import tpu as pltpu

N = 100000
G = 128
BLK = 2000


def _dense12_body(s_ref, w1_ref, b1_ref, w2_ref, o_ref):
    s = s_ref[...]
    h = jnp.maximum(s * w1_ref[...] + b1_ref[...], 0.0)
    o_ref[...] = jnp.dot(h, w2_ref[...], preferred_element_type=jnp.float32)


def _dense_mid_body(agg_ref, h_ref, inv_ref, b_ref, w_ref, o_ref):
    a = agg_ref[...] + h_ref[...] * inv_ref[...] + b_ref[...]
    o_ref[...] = jnp.dot(jnp.maximum(a, 0.0), w_ref[...],
                         preferred_element_type=jnp.float32)


def _dense_last_body(agg_ref, h_ref, inv_ref, b_ref, o_ref):
    a = agg_ref[...] + h_ref[...] * inv_ref[...] + b_ref[...]
    o_ref[...] = jnp.maximum(a, 0.0)


def _row_spec(width):
    return pl.BlockSpec((BLK, width), lambda i: (i, 0))


def _full_spec(shape):
    return pl.BlockSpec(shape, lambda i: tuple(0 for _ in shape))


def _dense12(s, W1, b1, W2):
    return pl.pallas_call(
        _dense12_body,
        grid=(N // BLK,),
        in_specs=[_row_spec(1), _full_spec((1, 128)), _full_spec((1, 128)),
                  _full_spec((128, 96))],
        out_specs=_row_spec(96),
        out_shape=jax.ShapeDtypeStruct((N, 96), jnp.float32),
    )(s, W1, b1.reshape(1, 128), W2)


def _dense_mid(agg, h, invdeg, b, Wnext, w_in, w_out):
    return pl.pallas_call(
        _dense_mid_body,
        grid=(N // BLK,),
        in_specs=[_row_spec(w_in), _row_spec(w_in), _row_spec(1),
                  _full_spec((1, w_in)), _full_spec((w_in, w_out))],
        out_specs=_row_spec(w_out),
        out_shape=jax.ShapeDtypeStruct((N, w_out), jnp.float32),
    )(agg, h, invdeg, b.reshape(1, w_in), Wnext)


def _dense_last(agg, h, invdeg, b, w_in):
    return pl.pallas_call(
        _dense_last_body,
        grid=(N // BLK,),
        in_specs=[_row_spec(w_in), _row_spec(w_in), _row_spec(1),
                  _full_spec((1, w_in))],
        out_specs=_row_spec(w_in),
        out_shape=jax.ShapeDtypeStruct((N, w_in), jnp.float32),
    )(agg, h, invdeg, b.reshape(1, w_in))


def kernel(x, edge_index, edge_weight, batch, W1, b1, W2, b2, W3, b3, W4, b4,
           Wl1, bl1, Wl2, bl2):
    src = edge_index[0]
    dst = edge_index[1]
    ew = edge_weight

    deg = jnp.ones((N,), jnp.float32).at[dst].add(ew)
    dis = jax.lax.rsqrt(deg)
    invdeg = (1.0 / deg)[:, None]

    norm = dis[src] * ew * dis[dst]

    xs = x[:, 0]
    s1 = jnp.zeros((N,), jnp.float32).at[dst].add(norm * xs[src]) + xs / deg
    h2 = _dense12(s1[:, None], W1, b1, W2)

    def agg(h, width):
        return jnp.zeros((N, width), jnp.float32).at[dst].add(
            norm[:, None] * h[src])

    h3 = _dense_mid(agg(h2, 96), h2, invdeg, b2, W3, 96, 64)
    h4 = _dense_mid(agg(h3, 64), h3, invdeg, b3, W4, 64, 32)
    out4 = _dense_last(agg(h4, 32), h4, invdeg, b4, 32)

    pooled = jax.ops.segment_max(out4, batch, num_segments=G)
    hh = jax.nn.relu(pooled @ Wl1 + bl1)
    return hh @ Wl2 + bl2


# speedup vs baseline: 2.4447x; 2.4447x over previous
import functools

import jax
import jax.numpy as jnp
from jax import lax
from jax.experimental import pallas as pl
from jax.experimental.pallas import tpu as pltpu
from jax.experimental.pallas import tpu_sc as plsc

N = 100000
G = 128
BLK = 2000

E = 1600000
EP = 1638400
GP = EP // (8 * 128)
GPT = GP // 32
SG = 5
NPAD = 100096
NSUB = NPAD // 16

_SC_MESH = plsc.VectorSubcoreMesh(core_axis_name="c", subcore_axis_name="s")


def _deg_body(dst_hbm, ew_hbm, zeros_hbm, out_hbm, idx_v, val_v, stage_v,
              acc_sh, sem):
    c = lax.axis_index("c")
    s = lax.axis_index("s")
    t = s * 2 + c

    pltpu.sync_copy(zeros_hbm.at[pl.ds(s * NSUB, NSUB)], stage_v)
    pltpu.sync_copy(stage_v, acc_sh.at[pl.ds(s * NSUB, NSUB)])
    plsc.subcore_barrier()

    def stage(k, carry):
        g0 = t * GPT + k * SG
        pltpu.sync_copy(dst_hbm.at[pl.ds(g0, SG)], idx_v)
        pltpu.sync_copy(ew_hbm.at[pl.ds(g0, SG)], val_v)
        cps = [pltpu.async_copy(val_v.at[g, j], acc_sh.at[idx_v.at[g, j]],
                                sem, add=True)
               for g in range(SG) for j in range(8)]
        for cp in cps:
            cp.wait()
        return carry

    lax.fori_loop(0, GPT // SG, stage, 0)

    plsc.subcore_barrier()
    pltpu.sync_copy(acc_sh.at[pl.ds(s * NSUB, NSUB)], stage_v)
    pltpu.sync_copy(stage_v, out_hbm.at[pl.ds(c * NPAD + s * NSUB, NSUB)])


def _gather3_body(src_hbm, dst_hbm, dis_hbm, xs_hbm,
                  gs_hbm, gd_hbm, gx_hbm,
                  srcv, dstv, gsv, gdv, gxv, sem):
    c = lax.axis_index("c")
    s = lax.axis_index("s")
    t = s * 2 + c

    def stage(k, carry):
        g0 = t * GPT + k * SG
        pltpu.sync_copy(src_hbm.at[pl.ds(g0, SG)], srcv)
        pltpu.sync_copy(dst_hbm.at[pl.ds(g0, SG)], dstv)
        for g in range(SG):
            cps = []
            for j in range(8):
                cps.append(pltpu.async_copy(
                    dis_hbm.at[srcv.at[g, j]], gsv.at[g, j], sem))
                cps.append(pltpu.async_copy(
                    dis_hbm.at[dstv.at[g, j]], gdv.at[g, j], sem))
                cps.append(pltpu.async_copy(
                    xs_hbm.at[srcv.at[g, j]], gxv.at[g, j], sem))
            for cp in cps:
                cp.wait()
        pltpu.sync_copy(gsv, gs_hbm.at[pl.ds(g0, SG)])
        pltpu.sync_copy(gdv, gd_hbm.at[pl.ds(g0, SG)])
        pltpu.sync_copy(gxv, gx_hbm.at[pl.ds(g0, SG)])
        return carry

    lax.fori_loop(0, GPT // SG, stage, 0)


def _sc_gather3(src_rows, dst_rows, dis_pad, xs_pad):
    erows = jax.ShapeDtypeStruct((GP, 8, 128), jnp.float32)
    k = pl.kernel(
        _gather3_body,
        mesh=_SC_MESH,
        out_type=(erows, erows, erows),
        scratch_types=[
            pltpu.VMEM((SG, 8, 128), jnp.int32),
            pltpu.VMEM((SG, 8, 128), jnp.int32),
            pltpu.VMEM((SG, 8, 128), jnp.float32),
            pltpu.VMEM((SG, 8, 128), jnp.float32),
            pltpu.VMEM((SG, 8, 128), jnp.float32),
            pltpu.SemaphoreType.DMA,
        ],
    )
    return k(src_rows, dst_rows, dis_pad, xs_pad)


def _sc_deg(dst_rows, ew_rows, zeros_npad):
    k = pl.kernel(
        _deg_body,
        mesh=_SC_MESH,
        out_type=jax.ShapeDtypeStruct((2 * NPAD,), jnp.float32),
        scratch_types=[
            pltpu.VMEM((SG, 8, 128), jnp.int32),
            pltpu.VMEM((SG, 8, 128), jnp.float32),
            pltpu.VMEM((NSUB,), jnp.float32),
            pltpu.VMEM_SHARED((NPAD,), jnp.float32),
            pltpu.SemaphoreType.DMA,
        ],
    )
    return k(dst_rows, ew_rows, zeros_npad)


def _edge_norm_body(gs_ref, gd_ref, gx_ref, ew_ref, norm_ref, s1v_ref):
    nrm = gs_ref[...] * ew_ref[...] * gd_ref[...]
    norm_ref[...] = nrm
    s1v_ref[...] = nrm * gx_ref[...]


def _tc_edge_norm(gs, gd, gx, ew_rows):
    eblk = pl.BlockSpec((GP // 25, 8, 128), lambda i: (i, 0, 0))
    erows = jax.ShapeDtypeStruct((GP, 8, 128), jnp.float32)
    return pl.pallas_call(
        _edge_norm_body,
        grid=(25,),
        in_specs=[eblk, eblk, eblk, eblk],
        out_specs=(eblk, eblk),
        out_shape=(erows, erows),
    )(gs, gd, gx, ew_rows)


def _dis_body(p0_ref, p1_ref, dis_ref, inv_ref):
    deg = 1.0 + p0_ref[...] + p1_ref[...]
    dis_ref[...] = jax.lax.rsqrt(deg)
    inv_ref[...] = 1.0 / deg


def _tc_dis(degp):
    p0 = degp[:NPAD].reshape(NPAD // 128, 128)
    p1 = degp[NPAD:].reshape(NPAD // 128, 128)
    blk = pl.BlockSpec((NPAD // 128, 128), lambda: (0, 0))
    shp = jax.ShapeDtypeStruct((NPAD // 128, 128), jnp.float32)
    dis, inv = pl.pallas_call(
        _dis_body,
        in_specs=[blk, blk],
        out_specs=(blk, blk),
        out_shape=(shp, shp),
    )(p0, p1)
    return dis.reshape(NPAD), inv.reshape(NPAD)


def _dense12_body(p0_ref, p1_ref, x_ref, inv_ref, w1_ref, b1_ref, w2_ref,
                  o_ref):
    s = p0_ref[...] + p1_ref[...] + x_ref[...] * inv_ref[...]
    h = jnp.maximum(s * w1_ref[...] + b1_ref[...], 0.0)
    o_ref[...] = jnp.dot(h, w2_ref[...], preferred_element_type=jnp.float32)


def _dense_mid_body(agg_ref, h_ref, inv_ref, b_ref, w_ref, o_ref):
    a = agg_ref[...] + h_ref[...] * inv_ref[...] + b_ref[...]
    o_ref[...] = jnp.dot(jnp.maximum(a, 0.0), w_ref[...],
                         preferred_element_type=jnp.float32)


def _dense_last_body(agg_ref, h_ref, inv_ref, b_ref, o_ref):
    a = agg_ref[...] + h_ref[...] * inv_ref[...] + b_ref[...]
    o_ref[...] = jnp.maximum(a, 0.0)


def _row_spec(width):
    return pl.BlockSpec((BLK, width), lambda i: (i, 0))


def _full_spec(shape):
    return pl.BlockSpec(shape, lambda i: tuple(0 for _ in shape))


def _dense12(p0, p1, x, invdeg, W1, b1, W2):
    return pl.pallas_call(
        _dense12_body,
        grid=(N // BLK,),
        in_specs=[_row_spec(1), _row_spec(1), _row_spec(1), _row_spec(1),
                  _full_spec((1, 128)), _full_spec((1, 128)),
                  _full_spec((128, 96))],
        out_specs=_row_spec(96),
        out_shape=jax.ShapeDtypeStruct((N, 96), jnp.float32),
    )(p0, p1, x, invdeg, W1, b1.reshape(1, 128), W2)


def _dense_mid(agg, h, invdeg, b, Wnext, w_in, w_out):
    return pl.pallas_call(
        _dense_mid_body,
        grid=(N // BLK,),
        in_specs=[_row_spec(w_in), _row_spec(w_in), _row_spec(1),
                  _full_spec((1, w_in)), _full_spec((w_in, w_out))],
        out_specs=_row_spec(w_out),
        out_shape=jax.ShapeDtypeStruct((N, w_out), jnp.float32),
    )(agg, h, invdeg, b.reshape(1, w_in), Wnext)


def _dense_last(agg, h, invdeg, b, w_in):
    return pl.pallas_call(
        _dense_last_body,
        grid=(N // BLK,),
        in_specs=[_row_spec(w_in), _row_spec(w_in), _row_spec(1),
                  _full_spec((1, w_in))],
        out_specs=_row_spec(w_in),
        out_shape=jax.ShapeDtypeStruct((N, w_in), jnp.float32),
    )(agg, h, invdeg, b.reshape(1, w_in))


def kernel(x, edge_index, edge_weight, batch, W1, b1, W2, b2, W3, b3, W4, b4,
           Wl1, bl1, Wl2, bl2):
    src = edge_index[0]
    dst = edge_index[1]
    ew = edge_weight

    pad = EP - E
    dst_p = jnp.concatenate([dst, jnp.full((pad,), N, jnp.int32)])
    src_p = jnp.concatenate([src, jnp.full((pad,), N, jnp.int32)])
    ew_p = jnp.concatenate([ew, jnp.zeros((pad,), jnp.float32)])
    dst_rows = dst_p.reshape(GP, 8, 128)
    src_rows = src_p.reshape(GP, 8, 128)
    ew_rows = ew_p.reshape(GP, 8, 128)
    zeros_npad = jnp.zeros((NPAD,), jnp.float32)
    xs = x[:, 0]
    xs_pad = jnp.concatenate([xs, jnp.zeros((NPAD - N,), jnp.float32)])

    degp = _sc_deg(dst_rows, ew_rows, zeros_npad)
    dis_pad, inv_pad = _tc_dis(degp)
    invdeg = inv_pad[:N, None]

    gs, gd, gx = _sc_gather3(src_rows, dst_rows, dis_pad, xs_pad)
    norm_rows, s1v_rows = _tc_edge_norm(gs, gd, gx, ew_rows)
    s1p = _sc_deg(dst_rows, s1v_rows, zeros_npad)
    h2 = _dense12(s1p[:N, None], s1p[NPAD:NPAD + N, None], x,
                  invdeg, W1, b1, W2)
    norm = norm_rows.reshape(EP)[:E]

    def agg(h, width):
        return jnp.zeros((N, width), jnp.float32).at[dst].add(
            norm[:, None] * h[src])

    h3 = _dense_mid(agg(h2, 96), h2, invdeg, b2, W3, 96, 64)
    h4 = _dense_mid(agg(h3, 64), h3, invdeg, b3, W4, 64, 32)
    out4 = _dense_last(agg(h4, 32), h4, invdeg, b4, 32)

    pooled = jax.ops.segment_max(out4, batch, num_segments=G)
    hh = jax.nn.relu(pooled @ Wl1 + bl1)
    return hh @ Wl2 + bl2


# speedup vs baseline: 3.9011x; 1.5957x over previous
import functools

import jax
import jax.numpy as jnp
from jax import lax
from jax.experimental import pallas as pl
from jax.experimental.pallas import tpu as pltpu
from jax.experimental.pallas import tpu_sc as plsc

N = 100000
G = 128
BLK = 2000

E = 1600000
EP = 1638400
GP = EP // (8 * 128)
GPT = GP // 32
SG = 5
NPAD = 100096
NSUB = NPAD // 16

_SC_MESH = plsc.VectorSubcoreMesh(core_axis_name="c", subcore_axis_name="s")


def _deg_body(dst_hbm, ew_hbm, zeros_hbm, out_hbm, idx_v, val_v, stage_v,
              acc_sh, sem):
    c = lax.axis_index("c")
    s = lax.axis_index("s")
    t = s * 2 + c

    pltpu.sync_copy(zeros_hbm.at[pl.ds(s * NSUB, NSUB)], stage_v)
    pltpu.sync_copy(stage_v, acc_sh.at[pl.ds(s * NSUB, NSUB)])
    plsc.subcore_barrier()

    def stage(k, carry):
        g0 = t * GPT + k * SG
        pltpu.sync_copy(dst_hbm.at[pl.ds(g0, SG)], idx_v)
        pltpu.sync_copy(ew_hbm.at[pl.ds(g0, SG)], val_v)
        cps = [pltpu.async_copy(val_v.at[g, j], acc_sh.at[idx_v.at[g, j]],
                                sem, add=True)
               for g in range(SG) for j in range(8)]
        for cp in cps:
            cp.wait()
        return carry

    lax.fori_loop(0, GPT // SG, stage, 0)

    plsc.subcore_barrier()
    pltpu.sync_copy(acc_sh.at[pl.ds(s * NSUB, NSUB)], stage_v)
    pltpu.sync_copy(stage_v, out_hbm.at[pl.ds(c * NPAD + s * NSUB, NSUB)])


HALF = 50176
OUTROWS = 2 * HALF
ER2 = EP // 128
RPT2 = ER2 // 32
SG2 = 4
SROWS = SG2 * 8
PTN = HALF // 16
CPB = 392
RING = 4


def _agg32_body(dst_hbm, src_hbm, nrm_hbm, h_hbm, zeros_hbm, out_hbm,
                dstv, srcv, nrmv, idxr, rows_r, cpb_v, acc_sh, sem, gsem):
    c = lax.axis_index("c")
    s = lax.axis_index("s")
    lo = c * HALF

    pltpu.sync_copy(zeros_hbm, cpb_v)
    for b in range(PTN // CPB):
        pltpu.sync_copy(cpb_v, acc_sh.at[pl.ds(s * PTN + b * CPB, CPB)])
    plsc.subcore_barrier()

    def prep_and_fire(rr, ring):
        for u in range(8):
            d16 = dstv[rr, pl.ds(u * 16, 16)]
            mask = (d16 >= lo) & (d16 < lo + HALF)
            idxr[ring, pl.ds(u * 16, 16)] = jnp.where(
                mask, d16 - lo, jnp.full((16,), HALF, jnp.int32))
        return pltpu.async_copy(h_hbm.at[srcv.at[rr]], rows_r.at[ring],
                                gsem)

    def consume(rr, ring):
        def scale(r, cc):
            nv = jnp.full((16,), nrmv[rr, pl.ds(r, 16)][0], jnp.float32)
            rows_r[ring, r, pl.ds(0, 16)] = (
                rows_r[ring, r, pl.ds(0, 16)] * nv)
            return cc

        lax.fori_loop(0, 128, scale, 0)
        pltpu.sync_copy(rows_r.at[ring], acc_sh.at[idxr.at[ring]],
                        add=True)

    def stage(k, carry):
        r0 = (s * (ER2 // 16) + k * SROWS)
        pltpu.sync_copy(dst_hbm.at[pl.ds(r0, SROWS)], dstv)
        pltpu.sync_copy(src_hbm.at[pl.ds(r0, SROWS)], srcv)
        pltpu.sync_copy(nrm_hbm.at[pl.ds(r0, SROWS)],
                        nrmv.at[pl.ds(0, SROWS)])
        cps = [prep_and_fire(rr, rr % RING) for rr in range(RING - 1)]
        for rr in range(SROWS):
            if rr + RING - 1 < SROWS:
                cps.append(prep_and_fire(rr + RING - 1,
                                         (rr + RING - 1) % RING))
            cps[rr].wait()
            consume(rr, rr % RING)
        return carry

    lax.fori_loop(0, (ER2 // 16) // SROWS, stage, 0)

    plsc.subcore_barrier()
    for b in range(PTN // CPB):
        r0 = s * PTN + b * CPB
        pltpu.sync_copy(acc_sh.at[pl.ds(r0, CPB)], cpb_v)
        pltpu.sync_copy(cpb_v, out_hbm.at[pl.ds(lo + r0, CPB)])


def _sc_agg32(dst2, src2, nrm2, h32_pad):
    zeros = jnp.zeros((CPB, 16), jnp.float32)
    k = pl.kernel(
        _agg32_body,
        mesh=_SC_MESH,
        compiler_params=pltpu.CompilerParams(use_tc_tiling_on_sc=False),
        out_type=jax.ShapeDtypeStruct((OUTROWS, 16), jnp.float32),
        scratch_types=[
            pltpu.VMEM((SROWS, 128), jnp.int32),
            pltpu.VMEM((SROWS, 128), jnp.int32),
            pltpu.VMEM((SROWS + 1, 128), jnp.float32),
            pltpu.VMEM((RING, 128), jnp.int32),
            pltpu.VMEM((RING, 128, 16), jnp.float32),
            pltpu.VMEM((CPB, 16), jnp.float32),
            pltpu.VMEM_SHARED((HALF + 16, 16), jnp.float32),
            pltpu.SemaphoreType.DMA,
            pltpu.SemaphoreType.DMA,
        ],
    )
    return k(dst2, src2, nrm2, h32_pad, zeros)


def _gather3_body(src_hbm, dst_hbm, dis_hbm, xs_hbm,
                  gs_hbm, gd_hbm, gx_hbm,
                  srcv, dstv, gsv, gdv, gxv, sem):
    c = lax.axis_index("c")
    s = lax.axis_index("s")
    t = s * 2 + c

    def stage(k, carry):
        g0 = t * GPT + k * SG
        pltpu.sync_copy(src_hbm.at[pl.ds(g0, SG)], srcv)
        pltpu.sync_copy(dst_hbm.at[pl.ds(g0, SG)], dstv)
        for g in range(SG):
            cps = []
            for j in range(8):
                cps.append(pltpu.async_copy(
                    dis_hbm.at[srcv.at[g, j]], gsv.at[g, j], sem))
                cps.append(pltpu.async_copy(
                    dis_hbm.at[dstv.at[g, j]], gdv.at[g, j], sem))
                cps.append(pltpu.async_copy(
                    xs_hbm.at[srcv.at[g, j]], gxv.at[g, j], sem))
            for cp in cps:
                cp.wait()
        pltpu.sync_copy(gsv, gs_hbm.at[pl.ds(g0, SG)])
        pltpu.sync_copy(gdv, gd_hbm.at[pl.ds(g0, SG)])
        pltpu.sync_copy(gxv, gx_hbm.at[pl.ds(g0, SG)])
        return carry

    lax.fori_loop(0, GPT // SG, stage, 0)


def _sc_gather3(src_rows, dst_rows, dis_pad, xs_pad):
    erows = jax.ShapeDtypeStruct((GP, 8, 128), jnp.float32)
    k = pl.kernel(
        _gather3_body,
        mesh=_SC_MESH,
        out_type=(erows, erows, erows),
        scratch_types=[
            pltpu.VMEM((SG, 8, 128), jnp.int32),
            pltpu.VMEM((SG, 8, 128), jnp.int32),
            pltpu.VMEM((SG, 8, 128), jnp.float32),
            pltpu.VMEM((SG, 8, 128), jnp.float32),
            pltpu.VMEM((SG, 8, 128), jnp.float32),
            pltpu.SemaphoreType.DMA,
        ],
    )
    return k(src_rows, dst_rows, dis_pad, xs_pad)


def _sc_deg(dst_rows, ew_rows, zeros_npad):
    k = pl.kernel(
        _deg_body,
        mesh=_SC_MESH,
        out_type=jax.ShapeDtypeStruct((2 * NPAD,), jnp.float32),
        scratch_types=[
            pltpu.VMEM((SG, 8, 128), jnp.int32),
            pltpu.VMEM((SG, 8, 128), jnp.float32),
            pltpu.VMEM((NSUB,), jnp.float32),
            pltpu.VMEM_SHARED((NPAD,), jnp.float32),
            pltpu.SemaphoreType.DMA,
        ],
    )
    return k(dst_rows, ew_rows, zeros_npad)


def _edge_norm_body(gs_ref, gd_ref, gx_ref, ew_ref, norm_ref, s1v_ref):
    nrm = gs_ref[...] * ew_ref[...] * gd_ref[...]
    norm_ref[...] = nrm
    s1v_ref[...] = nrm * gx_ref[...]


def _tc_edge_norm(gs, gd, gx, ew_rows):
    eblk = pl.BlockSpec((GP // 25, 8, 128), lambda i: (i, 0, 0))
    erows = jax.ShapeDtypeStruct((GP, 8, 128), jnp.float32)
    return pl.pallas_call(
        _edge_norm_body,
        grid=(25,),
        in_specs=[eblk, eblk, eblk, eblk],
        out_specs=(eblk, eblk),
        out_shape=(erows, erows),
    )(gs, gd, gx, ew_rows)


def _dis_body(p0_ref, p1_ref, dis_ref, inv_ref):
    deg = 1.0 + p0_ref[...] + p1_ref[...]
    dis_ref[...] = jax.lax.rsqrt(deg)
    inv_ref[...] = 1.0 / deg


def _tc_dis(degp):
    p0 = degp[:NPAD].reshape(NPAD // 128, 128)
    p1 = degp[NPAD:].reshape(NPAD // 128, 128)
    blk = pl.BlockSpec((NPAD // 128, 128), lambda: (0, 0))
    shp = jax.ShapeDtypeStruct((NPAD // 128, 128), jnp.float32)
    dis, inv = pl.pallas_call(
        _dis_body,
        in_specs=[blk, blk],
        out_specs=(blk, blk),
        out_shape=(shp, shp),
    )(p0, p1)
    return dis.reshape(NPAD), inv.reshape(NPAD)


def _dense12_body(p0_ref, p1_ref, x_ref, inv_ref, w1_ref, b1_ref, w2_ref,
                  o_ref):
    s = p0_ref[...] + p1_ref[...] + x_ref[...] * inv_ref[...]
    h = jnp.maximum(s * w1_ref[...] + b1_ref[...], 0.0)
    o_ref[...] = jnp.dot(h, w2_ref[...], preferred_element_type=jnp.float32)


def _dense_mid_body(agg_ref, h_ref, inv_ref, b_ref, w_ref, o_ref):
    a = agg_ref[...] + h_ref[...] * inv_ref[...] + b_ref[...]
    o_ref[...] = jnp.dot(jnp.maximum(a, 0.0), w_ref[...],
                         preferred_element_type=jnp.float32)


def _dense_last_body(agg_ref, h_ref, inv_ref, b_ref, o_ref):
    a = agg_ref[...] + h_ref[...] * inv_ref[...] + b_ref[...]
    o_ref[...] = jnp.maximum(a, 0.0)


def _row_spec(width):
    return pl.BlockSpec((BLK, width), lambda i: (i, 0))


def _full_spec(shape):
    return pl.BlockSpec(shape, lambda i: tuple(0 for _ in shape))


def _dense12(p0, p1, x, invdeg, W1, b1, W2):
    return pl.pallas_call(
        _dense12_body,
        grid=(N // BLK,),
        in_specs=[_row_spec(1), _row_spec(1), _row_spec(1), _row_spec(1),
                  _full_spec((1, 128)), _full_spec((1, 128)),
                  _full_spec((128, 96))],
        out_specs=_row_spec(96),
        out_shape=jax.ShapeDtypeStruct((N, 96), jnp.float32),
    )(p0, p1, x, invdeg, W1, b1.reshape(1, 128), W2)


def _dense_mid(agg, h, invdeg, b, Wnext, w_in, w_out):
    return pl.pallas_call(
        _dense_mid_body,
        grid=(N // BLK,),
        in_specs=[_row_spec(w_in), _row_spec(w_in), _row_spec(1),
                  _full_spec((1, w_in)), _full_spec((w_in, w_out))],
        out_specs=_row_spec(w_out),
        out_shape=jax.ShapeDtypeStruct((N, w_out), jnp.float32),
    )(agg, h, invdeg, b.reshape(1, w_in), Wnext)


def _dense_last(agg, h, invdeg, b, w_in):
    return pl.pallas_call(
        _dense_last_body,
        grid=(N // BLK,),
        in_specs=[_row_spec(w_in), _row_spec(w_in), _row_spec(1),
                  _full_spec((1, w_in))],
        out_specs=_row_spec(w_in),
        out_shape=jax.ShapeDtypeStruct((N, w_in), jnp.float32),
    )(agg, h, invdeg, b.reshape(1, w_in))


def kernel(x, edge_index, edge_weight, batch, W1, b1, W2, b2, W3, b3, W4, b4,
           Wl1, bl1, Wl2, bl2):
    src = edge_index[0]
    dst = edge_index[1]
    ew = edge_weight

    pad = EP - E
    dst_p = jnp.concatenate([dst, jnp.full((pad,), N, jnp.int32)])
    src_p = jnp.concatenate([src, jnp.full((pad,), N, jnp.int32)])
    ew_p = jnp.concatenate([ew, jnp.zeros((pad,), jnp.float32)])
    dst_rows = dst_p.reshape(GP, 8, 128)
    src_rows = src_p.reshape(GP, 8, 128)
    ew_rows = ew_p.reshape(GP, 8, 128)
    zeros_npad = jnp.zeros((NPAD,), jnp.float32)
    xs = x[:, 0]
    xs_pad = jnp.concatenate([xs, jnp.zeros((NPAD - N,), jnp.float32)])

    degp = _sc_deg(dst_rows, ew_rows, zeros_npad)
    dis_pad, inv_pad = _tc_dis(degp)
    invdeg = inv_pad[:N, None]

    gs, gd, gx = _sc_gather3(src_rows, dst_rows, dis_pad, xs_pad)
    norm_rows, s1v_rows = _tc_edge_norm(gs, gd, gx, ew_rows)
    s1p = _sc_deg(dst_rows, s1v_rows, zeros_npad)
    h2 = _dense12(s1p[:N, None], s1p[NPAD:NPAD + N, None], x,
                  invdeg, W1, b1, W2)

    dst2 = dst_p.reshape(ER2, 128)
    src2 = src_p.reshape(ER2, 128)
    nrm2 = norm_rows.reshape(ER2, 128)

    def agg(h, width):
        cols = []
        for i in range(width // 16):
            hs = h[:, i * 16:(i + 1) * 16]
            hs_pad = jnp.concatenate(
                [hs, jnp.zeros((NPAD - N, 16), jnp.float32)])
            cols.append(_sc_agg32(dst2, src2, nrm2, hs_pad)[:N])
        return jnp.concatenate(cols, axis=1) if len(cols) > 1 else cols[0]

    h3 = _dense_mid(agg(h2, 96), h2, invdeg, b2, W3, 96, 64)
    h4 = _dense_mid(agg(h3, 64), h3, invdeg, b3, W4, 64, 32)
    out4 = _dense_last(agg(h4, 32), h4, invdeg, b4, 32)

    pooled = jax.ops.segment_max(out4, batch, num_segments=G)
    hh = jax.nn.relu(pooled @ Wl1 + bl1)
    return hh @ Wl2 + bl2


# speedup vs baseline: 3.9311x; 1.0077x over previous
import functools

import jax
import jax.numpy as jnp
from jax import lax
from jax.experimental import pallas as pl
from jax.experimental.pallas import tpu as pltpu
from jax.experimental.pallas import tpu_sc as plsc

N = 100000
G = 128
BLK = 2000

E = 1600000
EP = 1638400
GP = EP // (8 * 128)
GPT = GP // 32
SG = 5
NPAD = 100096
NSUB = NPAD // 16

_SC_MESH = plsc.VectorSubcoreMesh(core_axis_name="c", subcore_axis_name="s")


def _deg_body(dst_hbm, ew_hbm, zeros_hbm, out_hbm, idx_v, val_v, stage_v,
              acc_sh, sem):
    c = lax.axis_index("c")
    s = lax.axis_index("s")
    t = s * 2 + c

    pltpu.sync_copy(zeros_hbm.at[pl.ds(s * NSUB, NSUB)], stage_v)
    pltpu.sync_copy(stage_v, acc_sh.at[pl.ds(s * NSUB, NSUB)])
    plsc.subcore_barrier()

    def stage(k, carry):
        g0 = t * GPT + k * SG
        pltpu.sync_copy(dst_hbm.at[pl.ds(g0, SG)], idx_v)
        pltpu.sync_copy(ew_hbm.at[pl.ds(g0, SG)], val_v)
        cps = [pltpu.async_copy(val_v.at[g, j], acc_sh.at[idx_v.at[g, j]],
                                sem, add=True)
               for g in range(SG) for j in range(8)]
        for cp in cps:
            cp.wait()
        return carry

    lax.fori_loop(0, GPT // SG, stage, 0)

    plsc.subcore_barrier()
    pltpu.sync_copy(acc_sh.at[pl.ds(s * NSUB, NSUB)], stage_v)
    pltpu.sync_copy(stage_v, out_hbm.at[pl.ds(c * NPAD + s * NSUB, NSUB)])


HALF = 50176
OUTROWS = 2 * HALF
ER2 = EP // 128
RPT2 = ER2 // 32
SG2 = 4
SROWS = SG2 * 8
PTN = HALF // 16
CPB = 392
RING = 4


def _agg32_body(dst_hbm, src_hbm, nrm_hbm, h_hbm, zeros_hbm, out_hbm,
                dstv, srcv, nrmv, idxr, rows_r, cpb_v, acc_sh, sem, gsem):
    c = lax.axis_index("c")
    s = lax.axis_index("s")
    lo = c * HALF

    pltpu.sync_copy(zeros_hbm, cpb_v)
    for b in range(PTN // CPB):
        pltpu.sync_copy(cpb_v, acc_sh.at[pl.ds(s * PTN + b * CPB, CPB)])
    plsc.subcore_barrier()

    def prep_and_fire(rr, ring):
        for u in range(8):
            d16 = dstv[rr, pl.ds(u * 16, 16)]
            mask = (d16 >= lo) & (d16 < lo + HALF)
            idxr[ring, pl.ds(u * 16, 16)] = jnp.where(
                mask, d16 - lo, jnp.full((16,), HALF, jnp.int32))
        return pltpu.async_copy(h_hbm.at[srcv.at[rr]], rows_r.at[ring],
                                gsem)

    def consume(rr, ring):
        def scale(r, cc):
            nv = jnp.full((16,), nrmv[rr, pl.ds(r, 16)][0], jnp.float32)
            rows_r[ring, r, pl.ds(0, 16)] = (
                rows_r[ring, r, pl.ds(0, 16)] * nv)
            return cc

        lax.fori_loop(0, 128, scale, 0)
        pltpu.sync_copy(rows_r.at[ring], acc_sh.at[idxr.at[ring]],
                        add=True)

    def stage(k, carry):
        r0 = (s * (ER2 // 16) + k * SROWS)
        pltpu.sync_copy(dst_hbm.at[pl.ds(r0, SROWS)], dstv)
        pltpu.sync_copy(src_hbm.at[pl.ds(r0, SROWS)], srcv)
        pltpu.sync_copy(nrm_hbm.at[pl.ds(r0, SROWS)],
                        nrmv.at[pl.ds(0, SROWS)])
        cps = [prep_and_fire(rr, rr % RING) for rr in range(RING - 1)]
        for rr in range(SROWS):
            if rr + RING - 1 < SROWS:
                cps.append(prep_and_fire(rr + RING - 1,
                                         (rr + RING - 1) % RING))
            cps[rr].wait()
            consume(rr, rr % RING)
        return carry

    lax.fori_loop(0, (ER2 // 16) // SROWS, stage, 0)

    plsc.subcore_barrier()
    for b in range(PTN // CPB):
        r0 = s * PTN + b * CPB
        pltpu.sync_copy(acc_sh.at[pl.ds(r0, CPB)], cpb_v)
        pltpu.sync_copy(cpb_v, out_hbm.at[pl.ds(lo + r0, CPB)])


def _sc_agg32(dst2, src2, nrm2, h32_pad):
    zeros = jnp.zeros((CPB, 16), jnp.float32)
    k = pl.kernel(
        _agg32_body,
        mesh=_SC_MESH,
        compiler_params=pltpu.CompilerParams(use_tc_tiling_on_sc=False),
        out_type=jax.ShapeDtypeStruct((OUTROWS, 16), jnp.float32),
        scratch_types=[
            pltpu.VMEM((SROWS, 128), jnp.int32),
            pltpu.VMEM((SROWS, 128), jnp.int32),
            pltpu.VMEM((SROWS + 1, 128), jnp.float32),
            pltpu.VMEM((RING, 128), jnp.int32),
            pltpu.VMEM((RING, 128, 16), jnp.float32),
            pltpu.VMEM((CPB, 16), jnp.float32),
            pltpu.VMEM_SHARED((HALF + 16, 16), jnp.float32),
            pltpu.SemaphoreType.DMA,
            pltpu.SemaphoreType.DMA,
        ],
    )
    return k(dst2, src2, nrm2, h32_pad, zeros)


RPT = NPAD // 32
RB = 184


def _segmax_body(h_hbm, b_hbm, out_hbm, hv, bv, accv, sem):
    c = lax.axis_index("c")
    s = lax.axis_index("s")
    t = s * 2 + c

    ninf = jnp.full((16,), -jnp.inf, jnp.float32)

    def init(i, cc):
        accv[i >> 1, pl.ds((i & 1) * 16, 16)] = ninf
        return cc

    lax.fori_loop(0, 272, init, 0)

    def stage(k, cc):
        r0 = t * RPT + k * RB
        pltpu.sync_copy(h_hbm.at[pl.ds(r0, RB)], hv)
        pltpu.sync_copy(b_hbm.at[pl.ds(r0, RB)], bv.at[pl.ds(0, RB)])

        def row(i, c2):
            b = bv[pl.ds(i, 16)][0]
            for u in range(2):
                cur = accv[b, pl.ds(u * 16, 16)]
                val = hv[i, pl.ds(u * 16, 16)]
                accv[b, pl.ds(u * 16, 16)] = jnp.maximum(cur, val)
            return c2

        lax.fori_loop(0, RB, row, 0)
        return cc

    lax.fori_loop(0, RPT // RB, stage, 0)
    pltpu.sync_copy(accv.at[pl.ds(0, G)], out_hbm.at[t])


def _sc_segmax(h_pad, batch_pad):
    k = pl.kernel(
        _segmax_body,
        mesh=_SC_MESH,
        out_type=jax.ShapeDtypeStruct((32, G, 32), jnp.float32),
        scratch_types=[
            pltpu.VMEM((RB, 32), jnp.float32),
            pltpu.VMEM((RB + 16,), jnp.int32),
            pltpu.VMEM((136, 32), jnp.float32),
            pltpu.SemaphoreType.DMA,
        ],
    )
    return k(h_pad, batch_pad)


def _head_body(p_ref, wl1_ref, bl1_ref, wl2_ref, bl2_ref, o_ref):
    pooled = jnp.max(p_ref[...], axis=0)
    h = jnp.maximum(jnp.dot(pooled, wl1_ref[...],
                            preferred_element_type=jnp.float32)
                    + bl1_ref[...], 0.0)
    o_ref[...] = jnp.dot(h, wl2_ref[...],
                         preferred_element_type=jnp.float32) + bl2_ref[...]


def _tc_head(partials, Wl1, bl1, Wl2, bl2):
    return pl.pallas_call(
        _head_body,
        grid=(1,),
        in_specs=[_full_spec((32, G, 32)), _full_spec((32, 32)),
                  _full_spec((1, 32)), _full_spec((32, 2)),
                  _full_spec((1, 2))],
        out_specs=_full_spec((G, 2)),
        out_shape=jax.ShapeDtypeStruct((G, 2), jnp.float32),
    )(partials, Wl1, bl1.reshape(1, 32), Wl2, bl2.reshape(1, 2))


def _gather3_body(src_hbm, dst_hbm, dis_hbm, xs_hbm,
                  gs_hbm, gd_hbm, gx_hbm,
                  srcv, dstv, gsv, gdv, gxv, sem):
    c = lax.axis_index("c")
    s = lax.axis_index("s")
    t = s * 2 + c

    def stage(k, carry):
        g0 = t * GPT + k * SG
        pltpu.sync_copy(src_hbm.at[pl.ds(g0, SG)], srcv)
        pltpu.sync_copy(dst_hbm.at[pl.ds(g0, SG)], dstv)
        for g in range(SG):
            cps = []
            for j in range(8):
                cps.append(pltpu.async_copy(
                    dis_hbm.at[srcv.at[g, j]], gsv.at[g, j], sem))
                cps.append(pltpu.async_copy(
                    dis_hbm.at[dstv.at[g, j]], gdv.at[g, j], sem))
                cps.append(pltpu.async_copy(
                    xs_hbm.at[srcv.at[g, j]], gxv.at[g, j], sem))
            for cp in cps:
                cp.wait()
        pltpu.sync_copy(gsv, gs_hbm.at[pl.ds(g0, SG)])
        pltpu.sync_copy(gdv, gd_hbm.at[pl.ds(g0, SG)])
        pltpu.sync_copy(gxv, gx_hbm.at[pl.ds(g0, SG)])
        return carry

    lax.fori_loop(0, GPT // SG, stage, 0)


def _sc_gather3(src_rows, dst_rows, dis_pad, xs_pad):
    erows = jax.ShapeDtypeStruct((GP, 8, 128), jnp.float32)
    k = pl.kernel(
        _gather3_body,
        mesh=_SC_MESH,
        out_type=(erows, erows, erows),
        scratch_types=[
            pltpu.VMEM((SG, 8, 128), jnp.int32),
            pltpu.VMEM((SG, 8, 128), jnp.int32),
            pltpu.VMEM((SG, 8, 128), jnp.float32),
            pltpu.VMEM((SG, 8, 128), jnp.float32),
            pltpu.VMEM((SG, 8, 128), jnp.float32),
            pltpu.SemaphoreType.DMA,
        ],
    )
    return k(src_rows, dst_rows, dis_pad, xs_pad)


def _sc_deg(dst_rows, ew_rows, zeros_npad):
    k = pl.kernel(
        _deg_body,
        mesh=_SC_MESH,
        out_type=jax.ShapeDtypeStruct((2 * NPAD,), jnp.float32),
        scratch_types=[
            pltpu.VMEM((SG, 8, 128), jnp.int32),
            pltpu.VMEM((SG, 8, 128), jnp.float32),
            pltpu.VMEM((NSUB,), jnp.float32),
            pltpu.VMEM_SHARED((NPAD,), jnp.float32),
            pltpu.SemaphoreType.DMA,
        ],
    )
    return k(dst_rows, ew_rows, zeros_npad)


def _edge_norm_body(gs_ref, gd_ref, gx_ref, ew_ref, norm_ref, s1v_ref):
    nrm = gs_ref[...] * ew_ref[...] * gd_ref[...]
    norm_ref[...] = nrm
    s1v_ref[...] = nrm * gx_ref[...]


def _tc_edge_norm(gs, gd, gx, ew_rows):
    eblk = pl.BlockSpec((GP // 25, 8, 128), lambda i: (i, 0, 0))
    erows = jax.ShapeDtypeStruct((GP, 8, 128), jnp.float32)
    return pl.pallas_call(
        _edge_norm_body,
        grid=(25,),
        in_specs=[eblk, eblk, eblk, eblk],
        out_specs=(eblk, eblk),
        out_shape=(erows, erows),
    )(gs, gd, gx, ew_rows)


def _dis_body(p0_ref, p1_ref, dis_ref, inv_ref):
    deg = 1.0 + p0_ref[...] + p1_ref[...]
    dis_ref[...] = jax.lax.rsqrt(deg)
    inv_ref[...] = 1.0 / deg


def _tc_dis(degp):
    p0 = degp[:NPAD].reshape(NPAD // 128, 128)
    p1 = degp[NPAD:].reshape(NPAD // 128, 128)
    blk = pl.BlockSpec((NPAD // 128, 128), lambda: (0, 0))
    shp = jax.ShapeDtypeStruct((NPAD // 128, 128), jnp.float32)
    dis, inv = pl.pallas_call(
        _dis_body,
        in_specs=[blk, blk],
        out_specs=(blk, blk),
        out_shape=(shp, shp),
    )(p0, p1)
    return dis.reshape(NPAD), inv.reshape(NPAD)


def _dense12_body(p0_ref, p1_ref, x_ref, inv_ref, w1_ref, b1_ref, w2_ref,
                  o_ref):
    s = p0_ref[...] + p1_ref[...] + x_ref[...] * inv_ref[...]
    h = jnp.maximum(s * w1_ref[...] + b1_ref[...], 0.0)
    o_ref[...] = jnp.dot(h, w2_ref[...], preferred_element_type=jnp.float32)


def _dense_mid_body(agg_ref, h_ref, inv_ref, b_ref, w_ref, o_ref):
    a = agg_ref[...] + h_ref[...] * inv_ref[...] + b_ref[...]
    o_ref[...] = jnp.dot(jnp.maximum(a, 0.0), w_ref[...],
                         preferred_element_type=jnp.float32)


def _dense_last_body(agg_ref, h_ref, inv_ref, b_ref, o_ref):
    a = agg_ref[...] + h_ref[...] * inv_ref[...] + b_ref[...]
    o_ref[...] = jnp.maximum(a, 0.0)


def _row_spec(width):
    return pl.BlockSpec((BLK, width), lambda i: (i, 0))


def _full_spec(shape):
    return pl.BlockSpec(shape, lambda i: tuple(0 for _ in shape))


def _dense12(p0, p1, x, invdeg, W1, b1, W2):
    return pl.pallas_call(
        _dense12_body,
        grid=(N // BLK,),
        in_specs=[_row_spec(1), _row_spec(1), _row_spec(1), _row_spec(1),
                  _full_spec((1, 128)), _full_spec((1, 128)),
                  _full_spec((128, 96))],
        out_specs=_row_spec(96),
        out_shape=jax.ShapeDtypeStruct((N, 96), jnp.float32),
    )(p0, p1, x, invdeg, W1, b1.reshape(1, 128), W2)


def _dense_mid(agg, h, invdeg, b, Wnext, w_in, w_out):
    return pl.pallas_call(
        _dense_mid_body,
        grid=(N // BLK,),
        in_specs=[_row_spec(w_in), _row_spec(w_in), _row_spec(1),
                  _full_spec((1, w_in)), _full_spec((w_in, w_out))],
        out_specs=_row_spec(w_out),
        out_shape=jax.ShapeDtypeStruct((N, w_out), jnp.float32),
    )(agg, h, invdeg, b.reshape(1, w_in), Wnext)


def _dense_last(agg, h, invdeg, b, w_in):
    return pl.pallas_call(
        _dense_last_body,
        grid=(N // BLK,),
        in_specs=[_row_spec(w_in), _row_spec(w_in), _row_spec(1),
                  _full_spec((1, w_in))],
        out_specs=_row_spec(w_in),
        out_shape=jax.ShapeDtypeStruct((N, w_in), jnp.float32),
    )(agg, h, invdeg, b.reshape(1, w_in))


def kernel(x, edge_index, edge_weight, batch, W1, b1, W2, b2, W3, b3, W4, b4,
           Wl1, bl1, Wl2, bl2):
    src = edge_index[0]
    dst = edge_index[1]
    ew = edge_weight

    pad = EP - E
    dst_p = jnp.concatenate([dst, jnp.full((pad,), N, jnp.int32)])
    src_p = jnp.concatenate([src, jnp.full((pad,), N, jnp.int32)])
    ew_p = jnp.concatenate([ew, jnp.zeros((pad,), jnp.float32)])
    dst_rows = dst_p.reshape(GP, 8, 128)
    src_rows = src_p.reshape(GP, 8, 128)
    ew_rows = ew_p.reshape(GP, 8, 128)
    zeros_npad = jnp.zeros((NPAD,), jnp.float32)
    xs = x[:, 0]
    xs_pad = jnp.concatenate([xs, jnp.zeros((NPAD - N,), jnp.float32)])

    degp = _sc_deg(dst_rows, ew_rows, zeros_npad)
    dis_pad, inv_pad = _tc_dis(degp)
    invdeg = inv_pad[:N, None]

    gs, gd, gx = _sc_gather3(src_rows, dst_rows, dis_pad, xs_pad)
    norm_rows, s1v_rows = _tc_edge_norm(gs, gd, gx, ew_rows)
    s1p = _sc_deg(dst_rows, s1v_rows, zeros_npad)
    h2 = _dense12(s1p[:N, None], s1p[NPAD:NPAD + N, None], x,
                  invdeg, W1, b1, W2)

    dst2 = dst_p.reshape(ER2, 128)
    src2 = src_p.reshape(ER2, 128)
    nrm2 = norm_rows.reshape(ER2, 128)

    def agg(h, width):
        cols = []
        for i in range(width // 16):
            hs = h[:, i * 16:(i + 1) * 16]
            hs_pad = jnp.concatenate(
                [hs, jnp.zeros((NPAD - N, 16), jnp.float32)])
            cols.append(_sc_agg32(dst2, src2, nrm2, hs_pad)[:N])
        return jnp.concatenate(cols, axis=1) if len(cols) > 1 else cols[0]

    h3 = _dense_mid(agg(h2, 96), h2, invdeg, b2, W3, 96, 64)
    h4 = _dense_mid(agg(h3, 64), h3, invdeg, b3, W4, 64, 32)
    out4 = _dense_last(agg(h4, 32), h4, invdeg, b4, 32)

    out4_pad = jnp.concatenate(
        [out4, jnp.zeros((NPAD - N, 32), jnp.float32)])
    batch_pad = jnp.concatenate(
        [batch, jnp.full((NPAD - N,), G, jnp.int32)])
    partials = _sc_segmax(out4_pad, batch_pad)
    return _tc_head(partials, Wl1, bl1, Wl2, bl2)


# speedup vs baseline: 3.9544x; 1.0059x over previous
import functools

import jax
import jax.numpy as jnp
from jax import lax
from jax.experimental import pallas as pl
from jax.experimental.pallas import tpu as pltpu
from jax.experimental.pallas import tpu_sc as plsc

N = 100000
G = 128
BLK = 2000

E = 1600000
EP = 1638400
GP = EP // (8 * 128)
GPT = GP // 32
SG = 5
NPAD = 100096
NSUB = NPAD // 16

_SC_MESH = plsc.VectorSubcoreMesh(core_axis_name="c", subcore_axis_name="s")


def _deg_body(dst_hbm, ew_hbm, zeros_hbm, out_hbm, idx_v, val_v, stage_v,
              acc_sh, sem):
    c = lax.axis_index("c")
    s = lax.axis_index("s")
    t = s * 2 + c

    pltpu.sync_copy(zeros_hbm.at[pl.ds(s * NSUB, NSUB)], stage_v)
    pltpu.sync_copy(stage_v, acc_sh.at[pl.ds(s * NSUB, NSUB)])
    plsc.subcore_barrier()

    def stage(k, carry):
        g0 = t * GPT + k * SG
        pltpu.sync_copy(dst_hbm.at[pl.ds(g0, SG)], idx_v)
        pltpu.sync_copy(ew_hbm.at[pl.ds(g0, SG)], val_v)
        cps = [pltpu.async_copy(val_v.at[g, j], acc_sh.at[idx_v.at[g, j]],
                                sem, add=True)
               for g in range(SG) for j in range(8)]
        for cp in cps:
            cp.wait()
        return carry

    lax.fori_loop(0, GPT // SG, stage, 0)

    plsc.subcore_barrier()
    pltpu.sync_copy(acc_sh.at[pl.ds(s * NSUB, NSUB)], stage_v)
    pltpu.sync_copy(stage_v, out_hbm.at[pl.ds(c * NPAD + s * NSUB, NSUB)])


HALF = 50176
OUTROWS = 2 * HALF
ER2 = EP // 128
RPT2 = ER2 // 32
SG2 = 4
SROWS = SG2 * 8
PTN = HALF // 16
CPB = 392
RING = 4


def _agg32_body(dst_hbm, src_hbm, nrm_hbm, h_hbm, zeros_hbm, out_hbm,
                dstv, srcv, nrmv, idxr, rows_r, cpb_v, acc_sh, sem, gsem):
    c = lax.axis_index("c")
    s = lax.axis_index("s")
    lo = c * HALF

    pltpu.sync_copy(zeros_hbm, cpb_v)
    for b in range(PTN // CPB):
        pltpu.sync_copy(cpb_v, acc_sh.at[pl.ds(s * PTN + b * CPB, CPB)])
    plsc.subcore_barrier()

    def prep_and_fire(rr, ring):
        for u in range(8):
            d16 = dstv[rr, pl.ds(u * 16, 16)]
            mask = (d16 >= lo) & (d16 < lo + HALF)
            idxr[ring, pl.ds(u * 16, 16)] = jnp.where(
                mask, d16 - lo, jnp.full((16,), HALF, jnp.int32))
        return pltpu.async_copy(h_hbm.at[srcv.at[rr]], rows_r.at[ring],
                                gsem)

    def consume(rr, ring):
        def scale(r, cc):
            nv = jnp.full((16,), nrmv[rr, pl.ds(r, 16)][0], jnp.float32)
            rows_r[ring, r, pl.ds(0, 16)] = (
                rows_r[ring, r, pl.ds(0, 16)] * nv)
            return cc

        lax.fori_loop(0, 128, scale, 0)
        return pltpu.async_copy(rows_r.at[ring], acc_sh.at[idxr.at[ring]],
                                sem, add=True)

    def stage(k, carry):
        r0 = (s * (ER2 // 16) + k * SROWS)
        pltpu.sync_copy(dst_hbm.at[pl.ds(r0, SROWS)], dstv)
        pltpu.sync_copy(src_hbm.at[pl.ds(r0, SROWS)], srcv)
        pltpu.sync_copy(nrm_hbm.at[pl.ds(r0, SROWS)],
                        nrmv.at[pl.ds(0, SROWS)])
        g_cps = [prep_and_fire(rr, rr % RING) for rr in range(RING - 1)]
        s_cps = [None] * SROWS
        for rr in range(SROWS):
            nxt = rr + RING - 1
            if nxt < SROWS:
                if nxt - RING >= 0:
                    s_cps[nxt - RING].wait()
                g_cps.append(prep_and_fire(nxt, nxt % RING))
            g_cps[rr].wait()
            s_cps[rr] = consume(rr, rr % RING)
        for rr in range(SROWS - RING, SROWS):
            s_cps[rr].wait()
        return carry

    lax.fori_loop(0, (ER2 // 16) // SROWS, stage, 0)

    plsc.subcore_barrier()
    for b in range(PTN // CPB):
        r0 = s * PTN + b * CPB
        pltpu.sync_copy(acc_sh.at[pl.ds(r0, CPB)], cpb_v)
        pltpu.sync_copy(cpb_v, out_hbm.at[pl.ds(lo + r0, CPB)])


def _sc_agg32(dst2, src2, nrm2, h32_pad):
    zeros = jnp.zeros((CPB, 16), jnp.float32)
    k = pl.kernel(
        _agg32_body,
        mesh=_SC_MESH,
        compiler_params=pltpu.CompilerParams(use_tc_tiling_on_sc=False),
        out_type=jax.ShapeDtypeStruct((OUTROWS, 16), jnp.float32),
        scratch_types=[
            pltpu.VMEM((SROWS, 128), jnp.int32),
            pltpu.VMEM((SROWS, 128), jnp.int32),
            pltpu.VMEM((SROWS + 1, 128), jnp.float32),
            pltpu.VMEM((RING, 128), jnp.int32),
            pltpu.VMEM((RING, 128, 16), jnp.float32),
            pltpu.VMEM((CPB, 16), jnp.float32),
            pltpu.VMEM_SHARED((HALF + 16, 16), jnp.float32),
            pltpu.SemaphoreType.DMA,
            pltpu.SemaphoreType.DMA,
        ],
    )
    return k(dst2, src2, nrm2, h32_pad, zeros)


RPT = NPAD // 32
RB = 184


def _segmax_body(h_hbm, b_hbm, out_hbm, hv, bv, accv, sem):
    c = lax.axis_index("c")
    s = lax.axis_index("s")
    t = s * 2 + c

    ninf = jnp.full((16,), -jnp.inf, jnp.float32)

    def init(i, cc):
        accv[i >> 1, pl.ds((i & 1) * 16, 16)] = ninf
        return cc

    lax.fori_loop(0, 272, init, 0)

    def stage(k, cc):
        r0 = t * RPT + k * RB
        pltpu.sync_copy(h_hbm.at[pl.ds(r0, RB)], hv)
        pltpu.sync_copy(b_hbm.at[pl.ds(r0, RB)], bv.at[pl.ds(0, RB)])

        def row(i, c2):
            b = bv[pl.ds(i, 16)][0]
            for u in range(2):
                cur = accv[b, pl.ds(u * 16, 16)]
                val = hv[i, pl.ds(u * 16, 16)]
                accv[b, pl.ds(u * 16, 16)] = jnp.maximum(cur, val)
            return c2

        lax.fori_loop(0, RB, row, 0)
        return cc

    lax.fori_loop(0, RPT // RB, stage, 0)
    pltpu.sync_copy(accv.at[pl.ds(0, G)], out_hbm.at[t])


def _sc_segmax(h_pad, batch_pad):
    k = pl.kernel(
        _segmax_body,
        mesh=_SC_MESH,
        out_type=jax.ShapeDtypeStruct((32, G, 32), jnp.float32),
        scratch_types=[
            pltpu.VMEM((RB, 32), jnp.float32),
            pltpu.VMEM((RB + 16,), jnp.int32),
            pltpu.VMEM((136, 32), jnp.float32),
            pltpu.SemaphoreType.DMA,
        ],
    )
    return k(h_pad, batch_pad)


def _head_body(p_ref, wl1_ref, bl1_ref, wl2_ref, bl2_ref, o_ref):
    pooled = jnp.max(p_ref[...], axis=0)
    h = jnp.maximum(jnp.dot(pooled, wl1_ref[...],
                            preferred_element_type=jnp.float32)
                    + bl1_ref[...], 0.0)
    o_ref[...] = jnp.dot(h, wl2_ref[...],
                         preferred_element_type=jnp.float32) + bl2_ref[...]


def _tc_head(partials, Wl1, bl1, Wl2, bl2):
    return pl.pallas_call(
        _head_body,
        grid=(1,),
        in_specs=[_full_spec((32, G, 32)), _full_spec((32, 32)),
                  _full_spec((1, 32)), _full_spec((32, 2)),
                  _full_spec((1, 2))],
        out_specs=_full_spec((G, 2)),
        out_shape=jax.ShapeDtypeStruct((G, 2), jnp.float32),
    )(partials, Wl1, bl1.reshape(1, 32), Wl2, bl2.reshape(1, 2))


def _gather3_body(src_hbm, dst_hbm, dis_hbm, xs_hbm,
                  gs_hbm, gd_hbm, gx_hbm,
                  srcv, dstv, gsv, gdv, gxv, sem):
    c = lax.axis_index("c")
    s = lax.axis_index("s")
    t = s * 2 + c

    def stage(k, carry):
        g0 = t * GPT + k * SG
        pltpu.sync_copy(src_hbm.at[pl.ds(g0, SG)], srcv)
        pltpu.sync_copy(dst_hbm.at[pl.ds(g0, SG)], dstv)
        for g in range(SG):
            cps = []
            for j in range(8):
                cps.append(pltpu.async_copy(
                    dis_hbm.at[srcv.at[g, j]], gsv.at[g, j], sem))
                cps.append(pltpu.async_copy(
                    dis_hbm.at[dstv.at[g, j]], gdv.at[g, j], sem))
                cps.append(pltpu.async_copy(
                    xs_hbm.at[srcv.at[g, j]], gxv.at[g, j], sem))
            for cp in cps:
                cp.wait()
        pltpu.sync_copy(gsv, gs_hbm.at[pl.ds(g0, SG)])
        pltpu.sync_copy(gdv, gd_hbm.at[pl.ds(g0, SG)])
        pltpu.sync_copy(gxv, gx_hbm.at[pl.ds(g0, SG)])
        return carry

    lax.fori_loop(0, GPT // SG, stage, 0)


def _sc_gather3(src_rows, dst_rows, dis_pad, xs_pad):
    erows = jax.ShapeDtypeStruct((GP, 8, 128), jnp.float32)
    k = pl.kernel(
        _gather3_body,
        mesh=_SC_MESH,
        out_type=(erows, erows, erows),
        scratch_types=[
            pltpu.VMEM((SG, 8, 128), jnp.int32),
            pltpu.VMEM((SG, 8, 128), jnp.int32),
            pltpu.VMEM((SG, 8, 128), jnp.float32),
            pltpu.VMEM((SG, 8, 128), jnp.float32),
            pltpu.VMEM((SG, 8, 128), jnp.float32),
            pltpu.SemaphoreType.DMA,
        ],
    )
    return k(src_rows, dst_rows, dis_pad, xs_pad)


def _sc_deg(dst_rows, ew_rows, zeros_npad):
    k = pl.kernel(
        _deg_body,
        mesh=_SC_MESH,
        out_type=jax.ShapeDtypeStruct((2 * NPAD,), jnp.float32),
        scratch_types=[
            pltpu.VMEM((SG, 8, 128), jnp.int32),
            pltpu.VMEM((SG, 8, 128), jnp.float32),
            pltpu.VMEM((NSUB,), jnp.float32),
            pltpu.VMEM_SHARED((NPAD,), jnp.float32),
            pltpu.SemaphoreType.DMA,
        ],
    )
    return k(dst_rows, ew_rows, zeros_npad)


def _edge_norm_body(gs_ref, gd_ref, gx_ref, ew_ref, norm_ref, s1v_ref):
    nrm = gs_ref[...] * ew_ref[...] * gd_ref[...]
    norm_ref[...] = nrm
    s1v_ref[...] = nrm * gx_ref[...]


def _tc_edge_norm(gs, gd, gx, ew_rows):
    eblk = pl.BlockSpec((GP // 25, 8, 128), lambda i: (i, 0, 0))
    erows = jax.ShapeDtypeStruct((GP, 8, 128), jnp.float32)
    return pl.pallas_call(
        _edge_norm_body,
        grid=(25,),
        in_specs=[eblk, eblk, eblk, eblk],
        out_specs=(eblk, eblk),
        out_shape=(erows, erows),
    )(gs, gd, gx, ew_rows)


def _dis_body(p0_ref, p1_ref, dis_ref, inv_ref):
    deg = 1.0 + p0_ref[...] + p1_ref[...]
    dis_ref[...] = jax.lax.rsqrt(deg)
    inv_ref[...] = 1.0 / deg


def _tc_dis(degp):
    p0 = degp[:NPAD].reshape(NPAD // 128, 128)
    p1 = degp[NPAD:].reshape(NPAD // 128, 128)
    blk = pl.BlockSpec((NPAD // 128, 128), lambda: (0, 0))
    shp = jax.ShapeDtypeStruct((NPAD // 128, 128), jnp.float32)
    dis, inv = pl.pallas_call(
        _dis_body,
        in_specs=[blk, blk],
        out_specs=(blk, blk),
        out_shape=(shp, shp),
    )(p0, p1)
    return dis.reshape(NPAD), inv.reshape(NPAD)


def _dense12_body(p0_ref, p1_ref, x_ref, inv_ref, w1_ref, b1_ref, w2_ref,
                  o_ref):
    s = p0_ref[...] + p1_ref[...] + x_ref[...] * inv_ref[...]
    h = jnp.maximum(s * w1_ref[...] + b1_ref[...], 0.0)
    o_ref[...] = jnp.dot(h, w2_ref[...], preferred_element_type=jnp.float32)


def _dense_mid_body(agg_ref, h_ref, inv_ref, b_ref, w_ref, o_ref):
    a = agg_ref[...] + h_ref[...] * inv_ref[...] + b_ref[...]
    o_ref[...] = jnp.dot(jnp.maximum(a, 0.0), w_ref[...],
                         preferred_element_type=jnp.float32)


def _dense_last_body(agg_ref, h_ref, inv_ref, b_ref, o_ref):
    a = agg_ref[...] + h_ref[...] * inv_ref[...] + b_ref[...]
    o_ref[...] = jnp.maximum(a, 0.0)


def _row_spec(width):
    return pl.BlockSpec((BLK, width), lambda i: (i, 0))


def _full_spec(shape):
    return pl.BlockSpec(shape, lambda i: tuple(0 for _ in shape))


def _dense12(p0, p1, x, invdeg, W1, b1, W2):
    return pl.pallas_call(
        _dense12_body,
        grid=(N // BLK,),
        in_specs=[_row_spec(1), _row_spec(1), _row_spec(1), _row_spec(1),
                  _full_spec((1, 128)), _full_spec((1, 128)),
                  _full_spec((128, 96))],
        out_specs=_row_spec(96),
        out_shape=jax.ShapeDtypeStruct((N, 96), jnp.float32),
    )(p0, p1, x, invdeg, W1, b1.reshape(1, 128), W2)


def _dense_mid(agg, h, invdeg, b, Wnext, w_in, w_out):
    return pl.pallas_call(
        _dense_mid_body,
        grid=(N // BLK,),
        in_specs=[_row_spec(w_in), _row_spec(w_in), _row_spec(1),
                  _full_spec((1, w_in)), _full_spec((w_in, w_out))],
        out_specs=_row_spec(w_out),
        out_shape=jax.ShapeDtypeStruct((N, w_out), jnp.float32),
    )(agg, h, invdeg, b.reshape(1, w_in), Wnext)


def _dense_last(agg, h, invdeg, b, w_in):
    return pl.pallas_call(
        _dense_last_body,
        grid=(N // BLK,),
        in_specs=[_row_spec(w_in), _row_spec(w_in), _row_spec(1),
                  _full_spec((1, w_in))],
        out_specs=_row_spec(w_in),
        out_shape=jax.ShapeDtypeStruct((N, w_in), jnp.float32),
    )(agg, h, invdeg, b.reshape(1, w_in))


def kernel(x, edge_index, edge_weight, batch, W1, b1, W2, b2, W3, b3, W4, b4,
           Wl1, bl1, Wl2, bl2):
    src = edge_index[0]
    dst = edge_index[1]
    ew = edge_weight

    pad = EP - E
    dst_p = jnp.concatenate([dst, jnp.full((pad,), N, jnp.int32)])
    src_p = jnp.concatenate([src, jnp.full((pad,), N, jnp.int32)])
    ew_p = jnp.concatenate([ew, jnp.zeros((pad,), jnp.float32)])
    dst_rows = dst_p.reshape(GP, 8, 128)
    src_rows = src_p.reshape(GP, 8, 128)
    ew_rows = ew_p.reshape(GP, 8, 128)
    zeros_npad = jnp.zeros((NPAD,), jnp.float32)
    xs = x[:, 0]
    xs_pad = jnp.concatenate([xs, jnp.zeros((NPAD - N,), jnp.float32)])

    degp = _sc_deg(dst_rows, ew_rows, zeros_npad)
    dis_pad, inv_pad = _tc_dis(degp)
    invdeg = inv_pad[:N, None]

    gs, gd, gx = _sc_gather3(src_rows, dst_rows, dis_pad, xs_pad)
    norm_rows, s1v_rows = _tc_edge_norm(gs, gd, gx, ew_rows)
    s1p = _sc_deg(dst_rows, s1v_rows, zeros_npad)
    h2 = _dense12(s1p[:N, None], s1p[NPAD:NPAD + N, None], x,
                  invdeg, W1, b1, W2)

    dst2 = dst_p.reshape(ER2, 128)
    src2 = src_p.reshape(ER2, 128)
    nrm2 = norm_rows.reshape(ER2, 128)

    def agg(h, width):
        cols = []
        for i in range(width // 16):
            hs = h[:, i * 16:(i + 1) * 16]
            hs_pad = jnp.concatenate(
                [hs, jnp.zeros((NPAD - N, 16), jnp.float32)])
            cols.append(_sc_agg32(dst2, src2, nrm2, hs_pad)[:N])
        return jnp.concatenate(cols, axis=1) if len(cols) > 1 else cols[0]

    h3 = _dense_mid(agg(h2, 96), h2, invdeg, b2, W3, 96, 64)
    h4 = _dense_mid(agg(h3, 64), h3, invdeg, b3, W4, 64, 32)
    out4 = _dense_last(agg(h4, 32), h4, invdeg, b4, 32)

    out4_pad = jnp.concatenate(
        [out4, jnp.zeros((NPAD - N, 32), jnp.float32)])
    batch_pad = jnp.concatenate(
        [batch, jnp.full((NPAD - N,), G, jnp.int32)])
    partials = _sc_segmax(out4_pad, batch_pad)
    return _tc_head(partials, Wl1, bl1, Wl2, bl2)
